# Initial kernel scaffold; baseline (speedup 1.0000x reference)
import jax
import jax.numpy as jnp
from jax.experimental import pallas as pl


def kernel(xy, points, W1, b1, g1, be1, W2, b2, g2, be2):
    raise NotImplementedError("write your pallas kernel here")


# speedup vs baseline: 265.7055x; 265.7055x over previous
import functools

import jax
import jax.numpy as jnp
from jax import lax
from jax.experimental import pallas as pl
from jax.experimental.pallas import tpu as pltpu
from jax.experimental.pallas import tpu_sc as plsc

_F32 = jnp.float32
_I32 = jnp.int32


def _fps_body(xx_ref, yy_ref, nx_ref, ny_ref):
    B, N = xx_ref.shape
    NP = nx_ref.shape[1]
    xx = xx_ref[...]
    yy = yy_ref[...]
    iota = lax.broadcasted_iota(_I32, (B, N), 1)
    col = lax.broadcasted_iota(_I32, (B, NP), 1)

    def step(i, carry):
        distance, farthest, nx_a, ny_a = carry
        heref = (col == i).astype(_F32)
        onehot = iota == farthest
        cx = jnp.sum(jnp.where(onehot, xx, 0.0), axis=1, keepdims=True)
        cy = jnp.sum(jnp.where(onehot, yy, 0.0), axis=1, keepdims=True)
        nx_a = nx_a + heref * cx
        ny_a = ny_a + heref * cy
        dx = xx - cx
        dy = yy - cy
        dist = dx * dx + dy * dy
        distance = jnp.minimum(distance, dist)
        m = jnp.max(distance, axis=1, keepdims=True)
        sel = jnp.where(distance == m, iota, N)
        farthest = jnp.min(sel, axis=1, keepdims=True)
        return distance, farthest, nx_a, ny_a

    distance0 = jnp.full((B, N), 1e10, dtype=_F32)
    farthest0 = jnp.zeros((B, 1), dtype=_I32)
    nxy0 = jnp.zeros((B, NP), dtype=_F32)
    _, _, nx, ny = lax.fori_loop(
        0, NP, step, (distance0, farthest0, nxy0, nxy0))
    nx_ref[...] = nx
    ny_ref[...] = ny


def _fps_call(xx, yy, NP):
    B, N = xx.shape
    return pl.pallas_call(
        _fps_body,
        out_shape=[
            jax.ShapeDtypeStruct((B, NP), _F32),
            jax.ShapeDtypeStruct((B, NP), _F32),
        ],
    )(xx, yy)


def _knn_body(xxs_ref, yys_ref, nx_ref, ny_ref, out_ref, *, K):
    N = xxs_ref.shape[1]
    NP = nx_ref.shape[2]
    px = xxs_ref[0]
    py = yys_ref[0]
    sx = nx_ref[0]
    sy = ny_ref[0]
    dx = px - sx
    dy = py - sy
    dist = dx * dx + dy * dy
    i0 = lax.broadcasted_iota(_I32, (N, NP), 0)
    for k in range(K):
        m = jnp.min(dist, axis=0, keepdims=True)
        sel = jnp.where(dist == m, i0, N)
        j = jnp.min(sel, axis=0, keepdims=True)
        out_ref[0, pl.ds(k, 1), :] = j
        dist = jnp.where(i0 == j, jnp.float32(jnp.inf), dist)


def _knn_call(xxs, yys, nx, ny, K):
    B, N, _ = xxs.shape
    NP = nx.shape[2]
    return pl.pallas_call(
        functools.partial(_knn_body, K=K),
        grid=(B,),
        in_specs=[
            pl.BlockSpec((1, N, 1), lambda b: (b, 0, 0)),
            pl.BlockSpec((1, N, 1), lambda b: (b, 0, 0)),
            pl.BlockSpec((1, 1, NP), lambda b: (b, 0, 0)),
            pl.BlockSpec((1, 1, NP), lambda b: (b, 0, 0)),
        ],
        out_specs=pl.BlockSpec((1, K, NP), lambda b: (b, 0, 0)),
        out_shape=jax.ShapeDtypeStruct((B, K, NP), _I32),
    )(xxs, yys, nx, ny)


def _q_body(f_ref, w_ref, q_ref):
    f = f_ref[0]
    w = w_ref[...]
    q_ref[0] = lax.dot_general(f, w, (((1,), (1,)), ((), ())),
                               preferred_element_type=_F32)


def _q_call(feats, W1):
    B, N, C = feats.shape
    O = W1.shape[0]
    return pl.pallas_call(
        _q_body,
        grid=(B,),
        in_specs=[
            pl.BlockSpec((1, N, C), lambda b: (b, 0, 0)),
            pl.BlockSpec((O, C), lambda b: (0, 0)),
        ],
        out_specs=pl.BlockSpec((1, N, O), lambda b: (b, 0, 0)),
        out_shape=jax.ShapeDtypeStruct((B, N, O), _F32),
    )(feats, W1)


def _gather_rows(table, gidx):
    T = gidx.shape[0]
    D = table.shape[1]
    info = plsc.get_sparse_core_info()
    NW = info.num_cores * info.num_subcores
    per_w = T // NW
    CH = 128
    n_ch = per_w // CH
    mesh = plsc.VectorSubcoreMesh(core_axis_name="c", subcore_axis_name="s")

    @functools.partial(
        pl.kernel,
        mesh=mesh,
        out_type=jax.ShapeDtypeStruct((T, D), _F32),
        scratch_types=[
            pltpu.VMEM((CH,), _I32),
            pltpu.VMEM((CH, D), _F32),
            pltpu.SemaphoreType.DMA,
        ],
    )
    def k(table_hbm, idx_hbm, out_hbm, idx_v, rows_v, sem):
        nc = info.num_cores
        wid = lax.axis_index("s") * nc + lax.axis_index("c")
        base = wid * per_w

        def body(j, _):
            b0 = pl.multiple_of(base + j * CH, CH)
            pltpu.sync_copy(idx_hbm.at[pl.ds(b0, CH)], idx_v)
            pltpu.async_copy(table_hbm.at[idx_v], rows_v, sem).wait()
            pltpu.sync_copy(rows_v, out_hbm.at[pl.ds(b0, CH)])
            return 0

        lax.fori_loop(0, n_ch, body, 0)

    return k(table, gidx)


def _k5_body(g_ref, nxy_ref, w1c_ref, b1_ref, sum_ref, sq_ref):
    SCH, K, O = g_ref.shape[1:]
    g = g_ref[0]
    nxy = nxy_ref[0]
    c = lax.dot_general(nxy, w1c_ref[...], (((1,), (1,)), ((), ())),
                        preferred_element_type=_F32)
    x1 = g - c[:, None, :] + b1_ref[...][None]
    x1f = x1.reshape(SCH * K, O)
    sum_ref[0, 0] = jnp.sum(x1f, axis=0, keepdims=True)
    sq_ref[0, 0] = jnp.sum(x1f * x1f, axis=0, keepdims=True)


def _k5_call(G4, new_xy, W1c, b1_2d, SCH):
    B, NP, K, O = G4.shape
    NCH = NP // SCH
    return pl.pallas_call(
        _k5_body,
        grid=(B, NCH),
        in_specs=[
            pl.BlockSpec((1, SCH, K, O), lambda b, j: (b, j, 0, 0)),
            pl.BlockSpec((1, SCH, 2), lambda b, j: (b, j, 0)),
            pl.BlockSpec((O, 2), lambda b, j: (0, 0)),
            pl.BlockSpec((1, O), lambda b, j: (0, 0)),
        ],
        out_specs=[
            pl.BlockSpec((1, 1, 1, O), lambda b, j: (b, j, 0, 0)),
            pl.BlockSpec((1, 1, 1, O), lambda b, j: (b, j, 0, 0)),
        ],
        out_shape=[
            jax.ShapeDtypeStruct((B, NCH, 1, O), _F32),
            jax.ShapeDtypeStruct((B, NCH, 1, O), _F32),
        ],
    )(G4, new_xy, W1c, b1_2d)


def _k6_body(g_ref, nxy_ref, w1c_ref, b1_ref, s1_ref, t1_ref, w2_ref,
             b2_ref, x2_ref, sum_ref, sq_ref):
    SCH, K, O = g_ref.shape[1:]
    g = g_ref[0]
    nxy = nxy_ref[0]
    c = lax.dot_general(nxy, w1c_ref[...], (((1,), (1,)), ((), ())),
                        preferred_element_type=_F32)
    x1 = g - c[:, None, :] + b1_ref[...][None]
    x1f = x1.reshape(SCH * K, O)
    h = jnp.maximum(x1f * s1_ref[...] + t1_ref[...], 0.0)
    x2 = lax.dot_general(h, w2_ref[...], (((1,), (1,)), ((), ())),
                         preferred_element_type=_F32) + b2_ref[...]
    x2_ref[0] = x2.reshape(SCH, K, O)
    sum_ref[0, 0] = jnp.sum(x2, axis=0, keepdims=True)
    sq_ref[0, 0] = jnp.sum(x2 * x2, axis=0, keepdims=True)


def _k6_call(G4, new_xy, W1c, b1_2d, s1, t1, W2, b2_2d, SCH):
    B, NP, K, O = G4.shape
    O2 = W2.shape[0]
    NCH = NP // SCH
    return pl.pallas_call(
        _k6_body,
        grid=(B, NCH),
        in_specs=[
            pl.BlockSpec((1, SCH, K, O), lambda b, j: (b, j, 0, 0)),
            pl.BlockSpec((1, SCH, 2), lambda b, j: (b, j, 0)),
            pl.BlockSpec((O, 2), lambda b, j: (0, 0)),
            pl.BlockSpec((1, O), lambda b, j: (0, 0)),
            pl.BlockSpec((1, O), lambda b, j: (0, 0)),
            pl.BlockSpec((1, O), lambda b, j: (0, 0)),
            pl.BlockSpec((O2, O), lambda b, j: (0, 0)),
            pl.BlockSpec((1, O2), lambda b, j: (0, 0)),
        ],
        out_specs=[
            pl.BlockSpec((1, SCH, K, O2), lambda b, j: (b, j, 0, 0)),
            pl.BlockSpec((1, 1, 1, O2), lambda b, j: (b, j, 0, 0)),
            pl.BlockSpec((1, 1, 1, O2), lambda b, j: (b, j, 0, 0)),
        ],
        out_shape=[
            jax.ShapeDtypeStruct((B, NP, K, O2), _F32),
            jax.ShapeDtypeStruct((B, NCH, 1, O2), _F32),
            jax.ShapeDtypeStruct((B, NCH, 1, O2), _F32),
        ],
    )(G4, new_xy, W1c, b1_2d, s1, t1, W2, b2_2d)


def _k7_body(x2_ref, s2_ref, t2_ref, out_ref):
    v = x2_ref[0]
    mx = jnp.max(v, axis=1)
    mn = jnp.min(v, axis=1)
    a = s2_ref[...]
    t = t2_ref[...]
    choose = jnp.where(a >= 0.0, mx, mn)
    out_ref[0] = jnp.maximum(choose * a + t, 0.0)


def _k7_call(x2, s2, t2):
    B, NP, K, O2 = x2.shape
    return pl.pallas_call(
        _k7_body,
        grid=(B,),
        in_specs=[
            pl.BlockSpec((1, NP, K, O2), lambda b: (b, 0, 0, 0)),
            pl.BlockSpec((1, O2), lambda b: (0, 0)),
            pl.BlockSpec((1, O2), lambda b: (0, 0)),
        ],
        out_specs=pl.BlockSpec((1, NP, O2), lambda b: (b, 0, 0)),
        out_shape=jax.ShapeDtypeStruct((B, NP, O2), _F32),
    )(x2, s2, t2)


def kernel(xy, points, W1, b1, g1, be1, W2, b2, g2, be2):
    B, N, _ = xy.shape
    D = points.shape[-1]
    NP = N // 4
    K = 16
    O = W1.shape[0]
    O2 = W2.shape[0]
    EPS = 1e-5

    xx = xy[:, :, 0]
    yy = xy[:, :, 1]
    nx, ny = _fps_call(xx, yy, NP)
    new_xy = jnp.stack([nx, ny], axis=-1)

    idxT = _knn_call(xx[:, :, None], yy[:, :, None], nx[:, None, :],
                     ny[:, None, :], K)
    idx = jnp.transpose(idxT, (0, 2, 1))

    feats = jnp.concatenate([xy, points], axis=-1)
    Q = _q_call(feats, W1)

    gidx = (idx + (jnp.arange(B, dtype=_I32) * N)[:, None, None]).reshape(-1)
    G = _gather_rows(Q.reshape(B * N, O), gidx)
    G4 = G.reshape(B, NP, K, O)

    W1c = W1[:, :2]
    b1_2d = b1[None, :]
    SCH = 128
    M = jnp.float32(B * NP * K)

    sum1, sq1 = _k5_call(G4, new_xy, W1c, b1_2d, SCH)
    mean1 = jnp.sum(sum1, axis=(0, 1)) / M
    var1 = jnp.maximum(jnp.sum(sq1, axis=(0, 1)) / M - mean1 * mean1, 0.0)
    s1 = g1[None, :] / jnp.sqrt(var1 + EPS)
    t1 = be1[None, :] - mean1 * s1

    x2, sum2, sq2 = _k6_call(G4, new_xy, W1c, b1_2d, s1, t1, W2,
                             b2[None, :], SCH)
    mean2 = jnp.sum(sum2, axis=(0, 1)) / M
    var2 = jnp.maximum(jnp.sum(sq2, axis=(0, 1)) / M - mean2 * mean2, 0.0)
    s2 = g2[None, :] / jnp.sqrt(var2 + EPS)
    t2 = be2[None, :] - mean2 * s2

    out = _k7_call(x2, s2, t2)
    return (new_xy, out)


# speedup vs baseline: 280.5194x; 1.0558x over previous
import functools

import jax
import jax.numpy as jnp
from jax import lax
from jax.experimental import pallas as pl
from jax.experimental.pallas import tpu as pltpu
from jax.experimental.pallas import tpu_sc as plsc

_F32 = jnp.float32
_I32 = jnp.int32


def _fps_body(xx_ref, yy_ref, nx_ref, ny_ref):
    B, N = xx_ref.shape
    NP = nx_ref.shape[1]
    xx = xx_ref[...]
    yy = yy_ref[...]
    xy2 = jnp.concatenate([xx, yy], axis=0)
    iota = lax.broadcasted_iota(_I32, (B, N), 1)
    col = lax.broadcasted_iota(_I32, (B, NP), 1)

    def step(i, carry):
        distance, farthest, nx_a, ny_a = carry
        heref = (col == i).astype(_F32)
        onehot = iota == farthest
        oh2 = jnp.concatenate([onehot, onehot], axis=0)
        red = jnp.sum(jnp.where(oh2, xy2, 0.0), axis=1, keepdims=True)
        cx = red[:B]
        cy = red[B:]
        nx_a = nx_a + heref * cx
        ny_a = ny_a + heref * cy
        dx = xx - cx
        dy = yy - cy
        dist = dx * dx + dy * dy
        distance = jnp.minimum(distance, dist)
        m = jnp.max(distance, axis=1, keepdims=True)
        sel = jnp.where(distance == m, iota, N)
        farthest = jnp.min(sel, axis=1, keepdims=True)
        return distance, farthest, nx_a, ny_a

    distance0 = jnp.full((B, N), 1e10, dtype=_F32)
    farthest0 = jnp.zeros((B, 1), dtype=_I32)
    nxy0 = jnp.zeros((B, NP), dtype=_F32)
    _, _, nx, ny = lax.fori_loop(
        0, NP, step, (distance0, farthest0, nxy0, nxy0))
    nx_ref[...] = nx
    ny_ref[...] = ny


def _fps_call(xx, yy, NP):
    B, N = xx.shape
    return pl.pallas_call(
        _fps_body,
        out_shape=[
            jax.ShapeDtypeStruct((B, NP), _F32),
            jax.ShapeDtypeStruct((B, NP), _F32),
        ],
    )(xx, yy)


def _knn_body(xxs_ref, yys_ref, nx_ref, ny_ref, out_ref, *, K):
    N = xxs_ref.shape[1]
    NP = nx_ref.shape[2]
    px = xxs_ref[0]
    py = yys_ref[0]
    sx = nx_ref[0]
    sy = ny_ref[0]
    dx = px - sx
    dy = py - sy
    dist = dx * dx + dy * dy
    i0 = lax.broadcasted_iota(_I32, (N, NP), 0)
    for k in range(K):
        m = jnp.min(dist, axis=0, keepdims=True)
        sel = jnp.where(dist == m, i0, N)
        j = jnp.min(sel, axis=0, keepdims=True)
        out_ref[0, pl.ds(k, 1), :] = j
        dist = jnp.where(i0 == j, jnp.float32(jnp.inf), dist)


def _knn_call(xxs, yys, nx, ny, K):
    B, N, _ = xxs.shape
    NP = nx.shape[2]
    return pl.pallas_call(
        functools.partial(_knn_body, K=K),
        grid=(B,),
        in_specs=[
            pl.BlockSpec((1, N, 1), lambda b: (b, 0, 0)),
            pl.BlockSpec((1, N, 1), lambda b: (b, 0, 0)),
            pl.BlockSpec((1, 1, NP), lambda b: (b, 0, 0)),
            pl.BlockSpec((1, 1, NP), lambda b: (b, 0, 0)),
        ],
        out_specs=pl.BlockSpec((1, K, NP), lambda b: (b, 0, 0)),
        out_shape=jax.ShapeDtypeStruct((B, K, NP), _I32),
    )(xxs, yys, nx, ny)


def _q_body(f_ref, w_ref, q_ref):
    f = f_ref[0]
    w = w_ref[...]
    q_ref[0] = lax.dot_general(f, w, (((1,), (1,)), ((), ())),
                               preferred_element_type=_F32)


def _q_call(feats, W1):
    B, N, C = feats.shape
    O = W1.shape[0]
    return pl.pallas_call(
        _q_body,
        grid=(B,),
        in_specs=[
            pl.BlockSpec((1, N, C), lambda b: (b, 0, 0)),
            pl.BlockSpec((O, C), lambda b: (0, 0)),
        ],
        out_specs=pl.BlockSpec((1, N, O), lambda b: (b, 0, 0)),
        out_shape=jax.ShapeDtypeStruct((B, N, O), _F32),
    )(feats, W1)


def _gather_rows(table, gidx):
    T = gidx.shape[0]
    D = table.shape[1]
    info = plsc.get_sparse_core_info()
    NW = info.num_cores * info.num_subcores
    per_w = T // NW
    CH = 128
    n_ch = per_w // CH
    mesh = plsc.VectorSubcoreMesh(core_axis_name="c", subcore_axis_name="s")

    @functools.partial(
        pl.kernel,
        mesh=mesh,
        out_type=jax.ShapeDtypeStruct((T, D), _F32),
        scratch_types=[
            pltpu.VMEM((CH,), _I32),
            pltpu.VMEM((CH, D), _F32),
            pltpu.SemaphoreType.DMA,
        ],
    )
    def k(table_hbm, idx_hbm, out_hbm, idx_v, rows_v, sem):
        nc = info.num_cores
        wid = lax.axis_index("s") * nc + lax.axis_index("c")
        base = wid * per_w

        def body(j, _):
            b0 = pl.multiple_of(base + j * CH, CH)
            pltpu.sync_copy(idx_hbm.at[pl.ds(b0, CH)], idx_v)
            pltpu.async_copy(table_hbm.at[idx_v], rows_v, sem).wait()
            pltpu.sync_copy(rows_v, out_hbm.at[pl.ds(b0, CH)])
            return 0

        lax.fori_loop(0, n_ch, body, 0)

    return k(table, gidx)


def _k5_body(g_ref, nxy_ref, w1c_ref, b1_ref, sum_ref, sq_ref):
    K, SCH, O = g_ref.shape[1:]
    g = g_ref[0]
    nxy = nxy_ref[0]
    c = lax.dot_general(nxy, w1c_ref[...], (((1,), (1,)), ((), ())),
                        preferred_element_type=_F32)
    x1 = g - c[None, :, :] + b1_ref[...][None]
    x1f = x1.reshape(K * SCH, O)
    sum_ref[0, 0] = jnp.sum(x1f, axis=0, keepdims=True)
    sq_ref[0, 0] = jnp.sum(x1f * x1f, axis=0, keepdims=True)


def _k5_call(G4, new_xy, W1c, b1_2d, SCH):
    B, K, NP, O = G4.shape
    NCH = NP // SCH
    return pl.pallas_call(
        _k5_body,
        grid=(B, NCH),
        in_specs=[
            pl.BlockSpec((1, K, SCH, O), lambda b, j: (b, 0, j, 0)),
            pl.BlockSpec((1, SCH, 2), lambda b, j: (b, j, 0)),
            pl.BlockSpec((O, 2), lambda b, j: (0, 0)),
            pl.BlockSpec((1, O), lambda b, j: (0, 0)),
        ],
        out_specs=[
            pl.BlockSpec((1, 1, 1, O), lambda b, j: (b, j, 0, 0)),
            pl.BlockSpec((1, 1, 1, O), lambda b, j: (b, j, 0, 0)),
        ],
        out_shape=[
            jax.ShapeDtypeStruct((B, NCH, 1, O), _F32),
            jax.ShapeDtypeStruct((B, NCH, 1, O), _F32),
        ],
    )(G4, new_xy, W1c, b1_2d)


def _k6_body(g_ref, nxy_ref, w1c_ref, b1_ref, s1_ref, t1_ref, w2_ref,
             b2_ref, mx_ref, mn_ref, sum_ref, sq_ref):
    K, SCH, O = g_ref.shape[1:]
    O2 = w2_ref.shape[0]
    g = g_ref[0]
    nxy = nxy_ref[0]
    c = lax.dot_general(nxy, w1c_ref[...], (((1,), (1,)), ((), ())),
                        preferred_element_type=_F32)
    x1 = g - c[None, :, :] + b1_ref[...][None]
    x1f = x1.reshape(K * SCH, O)
    h = jnp.maximum(x1f * s1_ref[...] + t1_ref[...], 0.0)
    x2 = lax.dot_general(h, w2_ref[...], (((1,), (1,)), ((), ())),
                         preferred_element_type=_F32) + b2_ref[...]
    sum_ref[0, 0] = jnp.sum(x2, axis=0, keepdims=True)
    sq_ref[0, 0] = jnp.sum(x2 * x2, axis=0, keepdims=True)
    x2r = x2.reshape(K, SCH, O2)
    mx_ref[0] = jnp.max(x2r, axis=0)
    mn_ref[0] = jnp.min(x2r, axis=0)


def _k6_call(G4, new_xy, W1c, b1_2d, s1, t1, W2, b2_2d, SCH):
    B, K, NP, O = G4.shape
    O2 = W2.shape[0]
    NCH = NP // SCH
    return pl.pallas_call(
        _k6_body,
        grid=(B, NCH),
        in_specs=[
            pl.BlockSpec((1, K, SCH, O), lambda b, j: (b, 0, j, 0)),
            pl.BlockSpec((1, SCH, 2), lambda b, j: (b, j, 0)),
            pl.BlockSpec((O, 2), lambda b, j: (0, 0)),
            pl.BlockSpec((1, O), lambda b, j: (0, 0)),
            pl.BlockSpec((1, O), lambda b, j: (0, 0)),
            pl.BlockSpec((1, O), lambda b, j: (0, 0)),
            pl.BlockSpec((O2, O), lambda b, j: (0, 0)),
            pl.BlockSpec((1, O2), lambda b, j: (0, 0)),
        ],
        out_specs=[
            pl.BlockSpec((1, SCH, O2), lambda b, j: (b, j, 0)),
            pl.BlockSpec((1, SCH, O2), lambda b, j: (b, j, 0)),
            pl.BlockSpec((1, 1, 1, O2), lambda b, j: (b, j, 0, 0)),
            pl.BlockSpec((1, 1, 1, O2), lambda b, j: (b, j, 0, 0)),
        ],
        out_shape=[
            jax.ShapeDtypeStruct((B, NP, O2), _F32),
            jax.ShapeDtypeStruct((B, NP, O2), _F32),
            jax.ShapeDtypeStruct((B, NCH, 1, O2), _F32),
            jax.ShapeDtypeStruct((B, NCH, 1, O2), _F32),
        ],
    )(G4, new_xy, W1c, b1_2d, s1, t1, W2, b2_2d)


def _k7_body(mx_ref, mn_ref, s2_ref, t2_ref, out_ref):
    mx = mx_ref[0]
    mn = mn_ref[0]
    a = s2_ref[...]
    t = t2_ref[...]
    choose = jnp.where(a >= 0.0, mx, mn)
    out_ref[0] = jnp.maximum(choose * a + t, 0.0)


def _k7_call(mx, mn, s2, t2):
    B, NP, O2 = mx.shape
    return pl.pallas_call(
        _k7_body,
        grid=(B,),
        in_specs=[
            pl.BlockSpec((1, NP, O2), lambda b: (b, 0, 0)),
            pl.BlockSpec((1, NP, O2), lambda b: (b, 0, 0)),
            pl.BlockSpec((1, O2), lambda b: (0, 0)),
            pl.BlockSpec((1, O2), lambda b: (0, 0)),
        ],
        out_specs=pl.BlockSpec((1, NP, O2), lambda b: (b, 0, 0)),
        out_shape=jax.ShapeDtypeStruct((B, NP, O2), _F32),
    )(mx, mn, s2, t2)


def kernel(xy, points, W1, b1, g1, be1, W2, b2, g2, be2):
    B, N, _ = xy.shape
    D = points.shape[-1]
    NP = N // 4
    K = 16
    O = W1.shape[0]
    O2 = W2.shape[0]
    EPS = 1e-5

    xx = xy[:, :, 0]
    yy = xy[:, :, 1]
    nx, ny = _fps_call(xx, yy, NP)
    new_xy = jnp.stack([nx, ny], axis=-1)

    idxT = _knn_call(xx[:, :, None], yy[:, :, None], nx[:, None, :],
                     ny[:, None, :], K)

    feats = jnp.concatenate([xy, points], axis=-1)
    Q = _q_call(feats, W1)

    gidx = (idxT + (jnp.arange(B, dtype=_I32) * N)[:, None, None]).reshape(-1)
    G = _gather_rows(Q.reshape(B * N, O), gidx)
    G4 = G.reshape(B, K, NP, O)

    W1c = W1[:, :2]
    b1_2d = b1[None, :]
    SCH = 128
    M = jnp.float32(B * NP * K)

    sum1, sq1 = _k5_call(G4, new_xy, W1c, b1_2d, SCH)
    mean1 = jnp.sum(sum1, axis=(0, 1)) / M
    var1 = jnp.maximum(jnp.sum(sq1, axis=(0, 1)) / M - mean1 * mean1, 0.0)
    s1 = g1[None, :] / jnp.sqrt(var1 + EPS)
    t1 = be1[None, :] - mean1 * s1

    mxp, mnp, sum2, sq2 = _k6_call(G4, new_xy, W1c, b1_2d, s1, t1, W2,
                                   b2[None, :], SCH)
    mean2 = jnp.sum(sum2, axis=(0, 1)) / M
    var2 = jnp.maximum(jnp.sum(sq2, axis=(0, 1)) / M - mean2 * mean2, 0.0)
    s2 = g2[None, :] / jnp.sqrt(var2 + EPS)
    t2 = be2[None, :] - mean2 * s2

    out = _k7_call(mxp, mnp, s2, t2)
    return (new_xy, out)


# speedup vs baseline: 308.1210x; 1.0984x over previous
import functools

import jax
import jax.numpy as jnp
from jax import lax
from jax.experimental import pallas as pl
from jax.experimental.pallas import tpu as pltpu
from jax.experimental.pallas import tpu_sc as plsc

_F32 = jnp.float32
_I32 = jnp.int32


def _fps_body(xx_ref, yy_ref, nx_ref, ny_ref):
    B, N = xx_ref.shape
    NP = nx_ref.shape[1]
    xx = xx_ref[...]
    yy = yy_ref[...]
    xy2 = jnp.concatenate([xx, yy], axis=0)
    iota = lax.broadcasted_iota(_I32, (B, N), 1)
    col = lax.broadcasted_iota(_I32, (B, NP), 1)

    def step(i, carry):
        distance, farthest, nx_a, ny_a = carry
        heref = (col == i).astype(_F32)
        onehot = iota == farthest
        oh2 = jnp.concatenate([onehot, onehot], axis=0)
        red = jnp.sum(jnp.where(oh2, xy2, 0.0), axis=1, keepdims=True)
        cx = red[:B]
        cy = red[B:]
        nx_a = nx_a + heref * cx
        ny_a = ny_a + heref * cy
        dx = xx - cx
        dy = yy - cy
        dist = dx * dx + dy * dy
        distance = jnp.minimum(distance, dist)
        m = jnp.max(distance, axis=1, keepdims=True)
        sel = jnp.where(distance == m, iota, N)
        farthest = jnp.min(sel, axis=1, keepdims=True)
        return distance, farthest, nx_a, ny_a

    distance0 = jnp.full((B, N), 1e10, dtype=_F32)
    farthest0 = jnp.zeros((B, 1), dtype=_I32)
    nxy0 = jnp.zeros((B, NP), dtype=_F32)
    _, _, nx, ny = lax.fori_loop(
        0, NP, step, (distance0, farthest0, nxy0, nxy0))
    nx_ref[...] = nx
    ny_ref[...] = ny


def _fps_call(xx, yy, NP):
    B, N = xx.shape
    return pl.pallas_call(
        _fps_body,
        out_shape=[
            jax.ShapeDtypeStruct((B, NP), _F32),
            jax.ShapeDtypeStruct((B, NP), _F32),
        ],
    )(xx, yy)


def _knn_body(xxs_ref, yys_ref, nx_ref, ny_ref, out_ref, *, K):
    N = xxs_ref.shape[1]
    NP = nx_ref.shape[2]
    px = xxs_ref[0]
    py = yys_ref[0]
    sx = nx_ref[0]
    sy = ny_ref[0]
    dx = px - sx
    dy = py - sy
    dist = dx * dx + dy * dy
    i0 = lax.broadcasted_iota(_I32, (N, NP), 0)
    for k in range(K):
        m = jnp.min(dist, axis=0, keepdims=True)
        sel = jnp.where(dist == m, i0, N)
        j = jnp.min(sel, axis=0, keepdims=True)
        out_ref[0, pl.ds(k, 1), :] = j
        dist = jnp.where(i0 == j, jnp.float32(jnp.inf), dist)


def _knn_call(xxs, yys, nx, ny, K):
    B, N, _ = xxs.shape
    NP = nx.shape[2]
    return pl.pallas_call(
        functools.partial(_knn_body, K=K),
        grid=(B,),
        in_specs=[
            pl.BlockSpec((1, N, 1), lambda b: (b, 0, 0)),
            pl.BlockSpec((1, N, 1), lambda b: (b, 0, 0)),
            pl.BlockSpec((1, 1, NP), lambda b: (b, 0, 0)),
            pl.BlockSpec((1, 1, NP), lambda b: (b, 0, 0)),
        ],
        out_specs=pl.BlockSpec((1, K, NP), lambda b: (b, 0, 0)),
        out_shape=jax.ShapeDtypeStruct((B, K, NP), _I32),
    )(xxs, yys, nx, ny)


def _gather_rows(table, gidx):
    T = gidx.shape[0]
    D = table.shape[1]
    info = plsc.get_sparse_core_info()
    NW = info.num_cores * info.num_subcores
    per_w = T // NW
    CH = 128
    n_ch = per_w // CH
    mesh = plsc.VectorSubcoreMesh(core_axis_name="c", subcore_axis_name="s")

    @functools.partial(
        pl.kernel,
        mesh=mesh,
        out_type=jax.ShapeDtypeStruct((T, D), _F32),
        scratch_types=[
            pltpu.VMEM((CH,), _I32),
            pltpu.VMEM((CH,), _I32),
            pltpu.VMEM((CH, D), _F32),
            pltpu.VMEM((CH, D), _F32),
            pltpu.SemaphoreType.DMA,
            pltpu.SemaphoreType.DMA,
            pltpu.SemaphoreType.DMA,
        ],
    )
    def k(table_hbm, idx_hbm, out_hbm, i0_v, i1_v, r0_v, r1_v,
          gsem, w0sem, w1sem):
        nc = info.num_cores
        wid = lax.axis_index("s") * nc + lax.axis_index("c")
        base = wid * per_w
        idxb = [i0_v, i1_v]
        rows = [r0_v, r1_v]
        wsem = [w0sem, w1sem]
        wpend = [None, None]
        for j in range(n_ch):
            b = j & 1
            if wpend[b] is not None:
                wpend[b].wait()
            b0 = pl.multiple_of(base + j * CH, CH)
            pltpu.sync_copy(idx_hbm.at[pl.ds(b0, CH)], idxb[b])
            pltpu.async_copy(table_hbm.at[idxb[b]], rows[b], gsem).wait()
            wpend[b] = pltpu.async_copy(rows[b], out_hbm.at[pl.ds(b0, CH)],
                                        wsem[b])
        for b in range(2):
            if wpend[b] is not None:
                wpend[b].wait()

    return k(table, gidx)


def _k5_body(g_ref, nxy_ref, w1_ref, w1c_ref, b1_ref, sum_ref, sq_ref):
    K, SCH, C = g_ref.shape[1:]
    O = w1_ref.shape[0]
    g = g_ref[0]
    nxy = nxy_ref[0]
    q = lax.dot_general(g.reshape(K * SCH, C), w1_ref[...],
                        (((1,), (1,)), ((), ())),
                        preferred_element_type=_F32)
    c = lax.dot_general(nxy, w1c_ref[...], (((1,), (1,)), ((), ())),
                        preferred_element_type=_F32)
    x1 = q.reshape(K, SCH, O) - c[None, :, :] + b1_ref[...][None]
    x1f = x1.reshape(K * SCH, O)
    sum_ref[0, 0] = jnp.sum(x1f, axis=0, keepdims=True)
    sq_ref[0, 0] = jnp.sum(x1f * x1f, axis=0, keepdims=True)


def _k5_call(G4, new_xy, W1, W1c, b1_2d, SCH):
    B, K, NP, C = G4.shape
    O = W1.shape[0]
    NCH = NP // SCH
    return pl.pallas_call(
        _k5_body,
        grid=(B, NCH),
        in_specs=[
            pl.BlockSpec((1, K, SCH, C), lambda b, j: (b, 0, j, 0)),
            pl.BlockSpec((1, SCH, 2), lambda b, j: (b, j, 0)),
            pl.BlockSpec((O, C), lambda b, j: (0, 0)),
            pl.BlockSpec((O, 2), lambda b, j: (0, 0)),
            pl.BlockSpec((1, O), lambda b, j: (0, 0)),
        ],
        out_specs=[
            pl.BlockSpec((1, 1, 1, O), lambda b, j: (b, j, 0, 0)),
            pl.BlockSpec((1, 1, 1, O), lambda b, j: (b, j, 0, 0)),
        ],
        out_shape=[
            jax.ShapeDtypeStruct((B, NCH, 1, O), _F32),
            jax.ShapeDtypeStruct((B, NCH, 1, O), _F32),
        ],
    )(G4, new_xy, W1, W1c, b1_2d)


def _k6_body(g_ref, nxy_ref, w1_ref, w1c_ref, b1_ref, s1_ref, t1_ref, w2_ref,
             b2_ref, mx_ref, mn_ref, sum_ref, sq_ref):
    K, SCH, C = g_ref.shape[1:]
    O = w1_ref.shape[0]
    O2 = w2_ref.shape[0]
    g = g_ref[0]
    nxy = nxy_ref[0]
    q = lax.dot_general(g.reshape(K * SCH, C), w1_ref[...],
                        (((1,), (1,)), ((), ())),
                        preferred_element_type=_F32)
    c = lax.dot_general(nxy, w1c_ref[...], (((1,), (1,)), ((), ())),
                        preferred_element_type=_F32)
    x1 = q.reshape(K, SCH, O) - c[None, :, :] + b1_ref[...][None]
    x1f = x1.reshape(K * SCH, O)
    h = jnp.maximum(x1f * s1_ref[...] + t1_ref[...], 0.0)
    x2 = lax.dot_general(h, w2_ref[...], (((1,), (1,)), ((), ())),
                         preferred_element_type=_F32) + b2_ref[...]
    sum_ref[0, 0] = jnp.sum(x2, axis=0, keepdims=True)
    sq_ref[0, 0] = jnp.sum(x2 * x2, axis=0, keepdims=True)
    x2r = x2.reshape(K, SCH, O2)
    mx_ref[0] = jnp.max(x2r, axis=0)
    mn_ref[0] = jnp.min(x2r, axis=0)


def _k6_call(G4, new_xy, W1, W1c, b1_2d, s1, t1, W2, b2_2d, SCH):
    B, K, NP, C = G4.shape
    O = W1.shape[0]
    O2 = W2.shape[0]
    NCH = NP // SCH
    return pl.pallas_call(
        _k6_body,
        grid=(B, NCH),
        in_specs=[
            pl.BlockSpec((1, K, SCH, C), lambda b, j: (b, 0, j, 0)),
            pl.BlockSpec((1, SCH, 2), lambda b, j: (b, j, 0)),
            pl.BlockSpec((O, C), lambda b, j: (0, 0)),
            pl.BlockSpec((O, 2), lambda b, j: (0, 0)),
            pl.BlockSpec((1, O), lambda b, j: (0, 0)),
            pl.BlockSpec((1, O), lambda b, j: (0, 0)),
            pl.BlockSpec((1, O), lambda b, j: (0, 0)),
            pl.BlockSpec((O2, O), lambda b, j: (0, 0)),
            pl.BlockSpec((1, O2), lambda b, j: (0, 0)),
        ],
        out_specs=[
            pl.BlockSpec((1, SCH, O2), lambda b, j: (b, j, 0)),
            pl.BlockSpec((1, SCH, O2), lambda b, j: (b, j, 0)),
            pl.BlockSpec((1, 1, 1, O2), lambda b, j: (b, j, 0, 0)),
            pl.BlockSpec((1, 1, 1, O2), lambda b, j: (b, j, 0, 0)),
        ],
        out_shape=[
            jax.ShapeDtypeStruct((B, NP, O2), _F32),
            jax.ShapeDtypeStruct((B, NP, O2), _F32),
            jax.ShapeDtypeStruct((B, NCH, 1, O2), _F32),
            jax.ShapeDtypeStruct((B, NCH, 1, O2), _F32),
        ],
    )(G4, new_xy, W1, W1c, b1_2d, s1, t1, W2, b2_2d)


def _k7_body(mx_ref, mn_ref, s2_ref, t2_ref, out_ref):
    mx = mx_ref[0]
    mn = mn_ref[0]
    a = s2_ref[...]
    t = t2_ref[...]
    choose = jnp.where(a >= 0.0, mx, mn)
    out_ref[0] = jnp.maximum(choose * a + t, 0.0)


def _k7_call(mx, mn, s2, t2):
    B, NP, O2 = mx.shape
    return pl.pallas_call(
        _k7_body,
        grid=(B,),
        in_specs=[
            pl.BlockSpec((1, NP, O2), lambda b: (b, 0, 0)),
            pl.BlockSpec((1, NP, O2), lambda b: (b, 0, 0)),
            pl.BlockSpec((1, O2), lambda b: (0, 0)),
            pl.BlockSpec((1, O2), lambda b: (0, 0)),
        ],
        out_specs=pl.BlockSpec((1, NP, O2), lambda b: (b, 0, 0)),
        out_shape=jax.ShapeDtypeStruct((B, NP, O2), _F32),
    )(mx, mn, s2, t2)


def kernel(xy, points, W1, b1, g1, be1, W2, b2, g2, be2):
    B, N, _ = xy.shape
    D = points.shape[-1]
    NP = N // 4
    K = 16
    O = W1.shape[0]
    O2 = W2.shape[0]
    EPS = 1e-5

    xx = xy[:, :, 0]
    yy = xy[:, :, 1]
    nx, ny = _fps_call(xx, yy, NP)
    new_xy = jnp.stack([nx, ny], axis=-1)

    idxT = _knn_call(xx[:, :, None], yy[:, :, None], nx[:, None, :],
                     ny[:, None, :], K)

    feats = jnp.concatenate([xy, points], axis=-1)
    C = feats.shape[-1]

    gidx = (idxT + (jnp.arange(B, dtype=_I32) * N)[:, None, None]).reshape(-1)
    G = _gather_rows(feats.reshape(B * N, C), gidx)
    G4 = G.reshape(B, K, NP, C)

    W1c = W1[:, :2]
    b1_2d = b1[None, :]
    SCH = 128
    M = jnp.float32(B * NP * K)

    sum1, sq1 = _k5_call(G4, new_xy, W1, W1c, b1_2d, SCH)
    mean1 = jnp.sum(sum1, axis=(0, 1)) / M
    var1 = jnp.maximum(jnp.sum(sq1, axis=(0, 1)) / M - mean1 * mean1, 0.0)
    s1 = g1[None, :] / jnp.sqrt(var1 + EPS)
    t1 = be1[None, :] - mean1 * s1

    mxp, mnp, sum2, sq2 = _k6_call(G4, new_xy, W1, W1c, b1_2d, s1, t1, W2,
                                   b2[None, :], SCH)
    mean2 = jnp.sum(sum2, axis=(0, 1)) / M
    var2 = jnp.maximum(jnp.sum(sq2, axis=(0, 1)) / M - mean2 * mean2, 0.0)
    s2 = g2[None, :] / jnp.sqrt(var2 + EPS)
    t2 = be2[None, :] - mean2 * s2

    out = _k7_call(mxp, mnp, s2, t2)
    return (new_xy, out)


# speedup vs baseline: 312.5737x; 1.0145x over previous
import functools

import jax
import jax.numpy as jnp
from jax import lax
from jax.experimental import pallas as pl
from jax.experimental.pallas import tpu as pltpu
from jax.experimental.pallas import tpu_sc as plsc

_F32 = jnp.float32
_I32 = jnp.int32


def _fps_body(xx_ref, yy_ref, nx_ref, ny_ref):
    B, N = xx_ref.shape
    NP = nx_ref.shape[1]
    xx = xx_ref[...]
    yy = yy_ref[...]
    xy2 = jnp.concatenate([xx, yy], axis=0)
    iota = lax.broadcasted_iota(_I32, (B, N), 1)
    col = lax.broadcasted_iota(_I32, (B, NP), 1)

    def step(i, carry):
        distance, farthest, nx_a, ny_a = carry
        heref = (col == i).astype(_F32)
        onehot = iota == farthest
        oh2 = jnp.concatenate([onehot, onehot], axis=0)
        red = jnp.sum(jnp.where(oh2, xy2, 0.0), axis=1, keepdims=True)
        cx = red[:B]
        cy = red[B:]
        nx_a = nx_a + heref * cx
        ny_a = ny_a + heref * cy
        dx = xx - cx
        dy = yy - cy
        dist = dx * dx + dy * dy
        distance = jnp.minimum(distance, dist)
        m = jnp.max(distance, axis=1, keepdims=True)
        sel = jnp.where(distance == m, iota, N)
        farthest = jnp.min(sel, axis=1, keepdims=True)
        return distance, farthest, nx_a, ny_a

    distance0 = jnp.full((B, N), 1e10, dtype=_F32)
    farthest0 = jnp.zeros((B, 1), dtype=_I32)
    nxy0 = jnp.zeros((B, NP), dtype=_F32)
    _, _, nx, ny = lax.fori_loop(
        0, NP, step, (distance0, farthest0, nxy0, nxy0))
    nx_ref[...] = nx
    ny_ref[...] = ny


def _fps_call(xx, yy, NP):
    B, N = xx.shape
    return pl.pallas_call(
        _fps_body,
        out_shape=[
            jax.ShapeDtypeStruct((B, NP), _F32),
            jax.ShapeDtypeStruct((B, NP), _F32),
        ],
    )(xx, yy)


def _knn_body(xxs_ref, yys_ref, nx_ref, ny_ref, out_ref, *, K):
    N = xxs_ref.shape[1]
    NP = nx_ref.shape[2]
    px = xxs_ref[0]
    py = yys_ref[0]
    sx = nx_ref[0]
    sy = ny_ref[0]
    dx = px - sx
    dy = py - sy
    dist = dx * dx + dy * dy
    i0 = lax.broadcasted_iota(_I32, (N, NP), 0)
    for k in range(K):
        m = jnp.min(dist, axis=0, keepdims=True)
        sel = jnp.where(dist == m, i0, N)
        j = jnp.min(sel, axis=0, keepdims=True)
        out_ref[0, pl.ds(k, 1), :] = j
        dist = jnp.where(i0 == j, jnp.float32(jnp.inf), dist)


def _knn_call(xxs, yys, nx, ny, K):
    B, N, _ = xxs.shape
    NP = nx.shape[2]
    return pl.pallas_call(
        functools.partial(_knn_body, K=K),
        grid=(B,),
        in_specs=[
            pl.BlockSpec((1, N, 1), lambda b: (b, 0, 0)),
            pl.BlockSpec((1, N, 1), lambda b: (b, 0, 0)),
            pl.BlockSpec((1, 1, NP), lambda b: (b, 0, 0)),
            pl.BlockSpec((1, 1, NP), lambda b: (b, 0, 0)),
        ],
        out_specs=pl.BlockSpec((1, K, NP), lambda b: (b, 0, 0)),
        out_shape=jax.ShapeDtypeStruct((B, K, NP), _I32),
    )(xxs, yys, nx, ny)


def _gather_rows(table, gidx):
    T = gidx.shape[0]
    D = table.shape[1]
    info = plsc.get_sparse_core_info()
    NW = info.num_cores * info.num_subcores
    per_w = T // NW
    CH = 128
    n_ch = per_w // CH
    mesh = plsc.VectorSubcoreMesh(core_axis_name="c", subcore_axis_name="s")

    @functools.partial(
        pl.kernel,
        mesh=mesh,
        out_type=jax.ShapeDtypeStruct((T, D), _F32),
        scratch_types=[
            pltpu.VMEM((CH,), _I32),
            pltpu.VMEM((CH,), _I32),
            pltpu.VMEM((CH, D), _F32),
            pltpu.VMEM((CH, D), _F32),
            pltpu.SemaphoreType.DMA,
            pltpu.SemaphoreType.DMA,
            pltpu.SemaphoreType.DMA,
            pltpu.SemaphoreType.DMA,
        ],
    )
    def k(table_hbm, idx_hbm, out_hbm, i0_v, i1_v, r0_v, r1_v,
          g0sem, g1sem, w0sem, w1sem):
        nc = info.num_cores
        wid = lax.axis_index("s") * nc + lax.axis_index("c")
        base = wid * per_w
        idxb = [i0_v, i1_v]
        rows = [r0_v, r1_v]
        gsem = [g0sem, g1sem]
        wsem = [w0sem, w1sem]
        gpend = [None, None]
        wpend = [None, None]
        boffs = [None, None]
        for j in range(n_ch):
            b = j & 1
            if wpend[b] is not None:
                wpend[b].wait()
            b0 = pl.multiple_of(base + j * CH, CH)
            pltpu.sync_copy(idx_hbm.at[pl.ds(b0, CH)], idxb[b])
            gpend[b] = pltpu.async_copy(table_hbm.at[idxb[b]], rows[b],
                                        gsem[b])
            boffs[b] = b0
            p = b ^ 1
            if gpend[p] is not None:
                gpend[p].wait()
                wpend[p] = pltpu.async_copy(
                    rows[p], out_hbm.at[pl.ds(boffs[p], CH)], wsem[p])
                gpend[p] = None
        last = (n_ch - 1) & 1
        if gpend[last] is not None:
            gpend[last].wait()
            wpend[last] = pltpu.async_copy(
                rows[last], out_hbm.at[pl.ds(boffs[last], CH)], wsem[last])
        for b in range(2):
            if wpend[b] is not None:
                wpend[b].wait()

    return k(table, gidx)


def _k5_body(g_ref, nxy_ref, w1_ref, w1c_ref, b1_ref, sum_ref, sq_ref):
    K, SCH, C = g_ref.shape[1:]
    O = w1_ref.shape[0]
    g = g_ref[0]
    nxy = nxy_ref[0]
    q = lax.dot_general(g.reshape(K * SCH, C), w1_ref[...],
                        (((1,), (1,)), ((), ())),
                        preferred_element_type=_F32)
    c = lax.dot_general(nxy, w1c_ref[...], (((1,), (1,)), ((), ())),
                        preferred_element_type=_F32)
    x1 = q.reshape(K, SCH, O) - c[None, :, :] + b1_ref[...][None]
    x1f = x1.reshape(K * SCH, O)
    sum_ref[0, 0] = jnp.sum(x1f, axis=0, keepdims=True)
    sq_ref[0, 0] = jnp.sum(x1f * x1f, axis=0, keepdims=True)


def _k5_call(G4, new_xy, W1, W1c, b1_2d, SCH):
    B, K, NP, C = G4.shape
    O = W1.shape[0]
    NCH = NP // SCH
    return pl.pallas_call(
        _k5_body,
        grid=(B, NCH),
        in_specs=[
            pl.BlockSpec((1, K, SCH, C), lambda b, j: (b, 0, j, 0)),
            pl.BlockSpec((1, SCH, 2), lambda b, j: (b, j, 0)),
            pl.BlockSpec((O, C), lambda b, j: (0, 0)),
            pl.BlockSpec((O, 2), lambda b, j: (0, 0)),
            pl.BlockSpec((1, O), lambda b, j: (0, 0)),
        ],
        out_specs=[
            pl.BlockSpec((1, 1, 1, O), lambda b, j: (b, j, 0, 0)),
            pl.BlockSpec((1, 1, 1, O), lambda b, j: (b, j, 0, 0)),
        ],
        out_shape=[
            jax.ShapeDtypeStruct((B, NCH, 1, O), _F32),
            jax.ShapeDtypeStruct((B, NCH, 1, O), _F32),
        ],
    )(G4, new_xy, W1, W1c, b1_2d)


def _k6_body(g_ref, nxy_ref, w1_ref, w1c_ref, b1_ref, s1_ref, t1_ref, w2_ref,
             b2_ref, mx_ref, mn_ref, sum_ref, sq_ref):
    K, SCH, C = g_ref.shape[1:]
    O = w1_ref.shape[0]
    O2 = w2_ref.shape[0]
    g = g_ref[0]
    nxy = nxy_ref[0]
    q = lax.dot_general(g.reshape(K * SCH, C), w1_ref[...],
                        (((1,), (1,)), ((), ())),
                        preferred_element_type=_F32)
    c = lax.dot_general(nxy, w1c_ref[...], (((1,), (1,)), ((), ())),
                        preferred_element_type=_F32)
    x1 = q.reshape(K, SCH, O) - c[None, :, :] + b1_ref[...][None]
    x1f = x1.reshape(K * SCH, O)
    h = jnp.maximum(x1f * s1_ref[...] + t1_ref[...], 0.0)
    x2 = lax.dot_general(h, w2_ref[...], (((1,), (1,)), ((), ())),
                         preferred_element_type=_F32) + b2_ref[...]
    sum_ref[0, 0] = jnp.sum(x2, axis=0, keepdims=True)
    sq_ref[0, 0] = jnp.sum(x2 * x2, axis=0, keepdims=True)
    x2r = x2.reshape(K, SCH, O2)
    mx_ref[0] = jnp.max(x2r, axis=0)
    mn_ref[0] = jnp.min(x2r, axis=0)


def _k6_call(G4, new_xy, W1, W1c, b1_2d, s1, t1, W2, b2_2d, SCH):
    B, K, NP, C = G4.shape
    O = W1.shape[0]
    O2 = W2.shape[0]
    NCH = NP // SCH
    return pl.pallas_call(
        _k6_body,
        grid=(B, NCH),
        in_specs=[
            pl.BlockSpec((1, K, SCH, C), lambda b, j: (b, 0, j, 0)),
            pl.BlockSpec((1, SCH, 2), lambda b, j: (b, j, 0)),
            pl.BlockSpec((O, C), lambda b, j: (0, 0)),
            pl.BlockSpec((O, 2), lambda b, j: (0, 0)),
            pl.BlockSpec((1, O), lambda b, j: (0, 0)),
            pl.BlockSpec((1, O), lambda b, j: (0, 0)),
            pl.BlockSpec((1, O), lambda b, j: (0, 0)),
            pl.BlockSpec((O2, O), lambda b, j: (0, 0)),
            pl.BlockSpec((1, O2), lambda b, j: (0, 0)),
        ],
        out_specs=[
            pl.BlockSpec((1, SCH, O2), lambda b, j: (b, j, 0)),
            pl.BlockSpec((1, SCH, O2), lambda b, j: (b, j, 0)),
            pl.BlockSpec((1, 1, 1, O2), lambda b, j: (b, j, 0, 0)),
            pl.BlockSpec((1, 1, 1, O2), lambda b, j: (b, j, 0, 0)),
        ],
        out_shape=[
            jax.ShapeDtypeStruct((B, NP, O2), _F32),
            jax.ShapeDtypeStruct((B, NP, O2), _F32),
            jax.ShapeDtypeStruct((B, NCH, 1, O2), _F32),
            jax.ShapeDtypeStruct((B, NCH, 1, O2), _F32),
        ],
    )(G4, new_xy, W1, W1c, b1_2d, s1, t1, W2, b2_2d)


def _k7_body(mx_ref, mn_ref, s2_ref, t2_ref, out_ref):
    mx = mx_ref[0]
    mn = mn_ref[0]
    a = s2_ref[...]
    t = t2_ref[...]
    choose = jnp.where(a >= 0.0, mx, mn)
    out_ref[0] = jnp.maximum(choose * a + t, 0.0)


def _k7_call(mx, mn, s2, t2):
    B, NP, O2 = mx.shape
    return pl.pallas_call(
        _k7_body,
        grid=(B,),
        in_specs=[
            pl.BlockSpec((1, NP, O2), lambda b: (b, 0, 0)),
            pl.BlockSpec((1, NP, O2), lambda b: (b, 0, 0)),
            pl.BlockSpec((1, O2), lambda b: (0, 0)),
            pl.BlockSpec((1, O2), lambda b: (0, 0)),
        ],
        out_specs=pl.BlockSpec((1, NP, O2), lambda b: (b, 0, 0)),
        out_shape=jax.ShapeDtypeStruct((B, NP, O2), _F32),
    )(mx, mn, s2, t2)


def kernel(xy, points, W1, b1, g1, be1, W2, b2, g2, be2):
    B, N, _ = xy.shape
    D = points.shape[-1]
    NP = N // 4
    K = 16
    O = W1.shape[0]
    O2 = W2.shape[0]
    EPS = 1e-5

    xx = xy[:, :, 0]
    yy = xy[:, :, 1]
    nx, ny = _fps_call(xx, yy, NP)
    new_xy = jnp.stack([nx, ny], axis=-1)

    idxT = _knn_call(xx[:, :, None], yy[:, :, None], nx[:, None, :],
                     ny[:, None, :], K)

    feats = jnp.concatenate([xy, points], axis=-1)
    C = feats.shape[-1]

    gidx = (idxT + (jnp.arange(B, dtype=_I32) * N)[:, None, None]).reshape(-1)
    G = _gather_rows(feats.reshape(B * N, C), gidx)
    G4 = G.reshape(B, K, NP, C)

    W1c = W1[:, :2]
    b1_2d = b1[None, :]
    SCH = 128
    M = jnp.float32(B * NP * K)

    sum1, sq1 = _k5_call(G4, new_xy, W1, W1c, b1_2d, SCH)
    mean1 = jnp.sum(sum1, axis=(0, 1)) / M
    var1 = jnp.maximum(jnp.sum(sq1, axis=(0, 1)) / M - mean1 * mean1, 0.0)
    s1 = g1[None, :] / jnp.sqrt(var1 + EPS)
    t1 = be1[None, :] - mean1 * s1

    mxp, mnp, sum2, sq2 = _k6_call(G4, new_xy, W1, W1c, b1_2d, s1, t1, W2,
                                   b2[None, :], SCH)
    mean2 = jnp.sum(sum2, axis=(0, 1)) / M
    var2 = jnp.maximum(jnp.sum(sq2, axis=(0, 1)) / M - mean2 * mean2, 0.0)
    s2 = g2[None, :] / jnp.sqrt(var2 + EPS)
    t2 = be2[None, :] - mean2 * s2

    out = _k7_call(mxp, mnp, s2, t2)
    return (new_xy, out)


# speedup vs baseline: 312.9604x; 1.0012x over previous
import functools

import jax
import jax.numpy as jnp
from jax import lax
from jax.experimental import pallas as pl
from jax.experimental.pallas import tpu as pltpu
from jax.experimental.pallas import tpu_sc as plsc

_F32 = jnp.float32
_I32 = jnp.int32


def _fps_body(xx_ref, yy_ref, nx_ref, ny_ref):
    B, N = xx_ref.shape
    NP = nx_ref.shape[1]
    xx = xx_ref[...]
    yy = yy_ref[...]
    xy2 = jnp.concatenate([xx, yy], axis=0)
    iota = lax.broadcasted_iota(_I32, (B, N), 1)
    col = lax.broadcasted_iota(_I32, (B, NP), 1)

    def step(i, carry):
        distance, farthest, nx_a, ny_a = carry
        heref = (col == i).astype(_F32)
        onehot = iota == farthest
        oh2 = jnp.concatenate([onehot, onehot], axis=0)
        red = jnp.sum(jnp.where(oh2, xy2, 0.0), axis=1, keepdims=True)
        cx = red[:B]
        cy = red[B:]
        nx_a = nx_a + heref * cx
        ny_a = ny_a + heref * cy
        dx = xx - cx
        dy = yy - cy
        dist = dx * dx + dy * dy
        distance = jnp.minimum(distance, dist)
        m = jnp.max(distance, axis=1, keepdims=True)
        sel = jnp.where(distance == m, iota, N)
        farthest = jnp.min(sel, axis=1, keepdims=True)
        return distance, farthest, nx_a, ny_a

    distance0 = jnp.full((B, N), 1e10, dtype=_F32)
    farthest0 = jnp.zeros((B, 1), dtype=_I32)
    nxy0 = jnp.zeros((B, NP), dtype=_F32)
    _, _, nx, ny = lax.fori_loop(
        0, NP, step, (distance0, farthest0, nxy0, nxy0))
    nx_ref[...] = nx
    ny_ref[...] = ny


def _fps_call(xx, yy, NP):
    B, N = xx.shape
    return pl.pallas_call(
        _fps_body,
        out_shape=[
            jax.ShapeDtypeStruct((B, NP), _F32),
            jax.ShapeDtypeStruct((B, NP), _F32),
        ],
    )(xx, yy)


def _knn_body(xxs_ref, yys_ref, nx_ref, ny_ref, out_ref, *, K):
    N = xxs_ref.shape[1]
    NP = nx_ref.shape[2]
    px = xxs_ref[0]
    py = yys_ref[0]
    sx = nx_ref[0]
    sy = ny_ref[0]
    dx = px - sx
    dy = py - sy
    dist = dx * dx + dy * dy
    i0 = lax.broadcasted_iota(_I32, (N, NP), 0)
    for k in range(K):
        m = jnp.min(dist, axis=0, keepdims=True)
        sel = jnp.where(dist == m, i0, N)
        j = jnp.min(sel, axis=0, keepdims=True)
        out_ref[0, pl.ds(k, 1), :] = j
        dist = jnp.where(i0 == j, jnp.float32(jnp.inf), dist)


def _knn_call(xxs, yys, nx, ny, K):
    B, N, _ = xxs.shape
    NP = nx.shape[2]
    return pl.pallas_call(
        functools.partial(_knn_body, K=K),
        grid=(B,),
        in_specs=[
            pl.BlockSpec((1, N, 1), lambda b: (b, 0, 0)),
            pl.BlockSpec((1, N, 1), lambda b: (b, 0, 0)),
            pl.BlockSpec((1, 1, NP), lambda b: (b, 0, 0)),
            pl.BlockSpec((1, 1, NP), lambda b: (b, 0, 0)),
        ],
        out_specs=pl.BlockSpec((1, K, NP), lambda b: (b, 0, 0)),
        out_shape=jax.ShapeDtypeStruct((B, K, NP), _I32),
    )(xxs, yys, nx, ny)


def _gather_rows(table, gidx):
    T = gidx.shape[0]
    D = table.shape[1]
    info = plsc.get_sparse_core_info()
    NW = info.num_cores * info.num_subcores
    per_w = T // NW
    CH = 128
    n_ch = per_w // CH
    mesh = plsc.VectorSubcoreMesh(core_axis_name="c", subcore_axis_name="s")

    @functools.partial(
        pl.kernel,
        mesh=mesh,
        out_type=jax.ShapeDtypeStruct((T, D), _F32),
        scratch_types=[
            pltpu.VMEM((CH,), _I32),
            pltpu.VMEM((CH,), _I32),
            pltpu.VMEM((CH, D), _F32),
            pltpu.VMEM((CH, D), _F32),
            pltpu.SemaphoreType.DMA,
            pltpu.SemaphoreType.DMA,
            pltpu.SemaphoreType.DMA,
            pltpu.SemaphoreType.DMA,
        ],
    )
    def k(table_hbm, idx_hbm, out_hbm, i0_v, i1_v, r0_v, r1_v,
          g0sem, g1sem, w0sem, w1sem):
        nc = info.num_cores
        wid = lax.axis_index("s") * nc + lax.axis_index("c")
        base = wid * per_w
        idxb = [i0_v, i1_v]
        rows = [r0_v, r1_v]
        gsem = [g0sem, g1sem]
        wsem = [w0sem, w1sem]
        gpend = [None, None]
        wpend = [None, None]
        boffs = [None, None]
        for j in range(n_ch):
            b = j & 1
            if wpend[b] is not None:
                wpend[b].wait()
            b0 = pl.multiple_of(base + j * CH, CH)
            pltpu.sync_copy(idx_hbm.at[pl.ds(b0, CH)], idxb[b])
            gpend[b] = pltpu.async_copy(table_hbm.at[idxb[b]], rows[b],
                                        gsem[b])
            boffs[b] = b0
            p = b ^ 1
            if gpend[p] is not None:
                gpend[p].wait()
                wpend[p] = pltpu.async_copy(
                    rows[p], out_hbm.at[pl.ds(boffs[p], CH)], wsem[p])
                gpend[p] = None
        last = (n_ch - 1) & 1
        if gpend[last] is not None:
            gpend[last].wait()
            wpend[last] = pltpu.async_copy(
                rows[last], out_hbm.at[pl.ds(boffs[last], CH)], wsem[last])
        for b in range(2):
            if wpend[b] is not None:
                wpend[b].wait()

    return k(table, gidx)


def _k5_body(g_ref, nxy_ref, w1_ref, w1c_ref, b1_ref, sum_ref, sq_ref):
    K, SCH, C = g_ref.shape[1:]
    O = w1_ref.shape[0]
    g = g_ref[0]
    nxy = nxy_ref[0]
    q = lax.dot_general(g.reshape(K * SCH, C), w1_ref[...],
                        (((1,), (1,)), ((), ())),
                        preferred_element_type=_F32)
    c = lax.dot_general(nxy, w1c_ref[...], (((1,), (1,)), ((), ())),
                        preferred_element_type=_F32)
    x1 = q.reshape(K, SCH, O) - c[None, :, :] + b1_ref[...][None]
    x1f = x1.reshape(K * SCH, O)
    sum_ref[0, 0] = jnp.sum(x1f, axis=0, keepdims=True)
    sq_ref[0, 0] = jnp.sum(x1f * x1f, axis=0, keepdims=True)


def _k5_call(G4, new_xy, W1, W1c, b1_2d, SCH):
    B, K, NP, C = G4.shape
    O = W1.shape[0]
    NCH = NP // SCH
    return pl.pallas_call(
        _k5_body,
        grid=(B, NCH),
        in_specs=[
            pl.BlockSpec((1, K, SCH, C), lambda b, j: (b, 0, j, 0)),
            pl.BlockSpec((1, SCH, 2), lambda b, j: (b, j, 0)),
            pl.BlockSpec((O, C), lambda b, j: (0, 0)),
            pl.BlockSpec((O, 2), lambda b, j: (0, 0)),
            pl.BlockSpec((1, O), lambda b, j: (0, 0)),
        ],
        out_specs=[
            pl.BlockSpec((1, 1, 1, O), lambda b, j: (b, j, 0, 0)),
            pl.BlockSpec((1, 1, 1, O), lambda b, j: (b, j, 0, 0)),
        ],
        out_shape=[
            jax.ShapeDtypeStruct((B, NCH, 1, O), _F32),
            jax.ShapeDtypeStruct((B, NCH, 1, O), _F32),
        ],
    )(G4, new_xy, W1, W1c, b1_2d)


def _k6_body(g_ref, nxy_ref, w1_ref, w1c_ref, b1_ref, s1_ref, t1_ref, w2_ref,
             b2_ref, mx_ref, mn_ref, sum_ref, sq_ref):
    K, SCH, C = g_ref.shape[1:]
    O = w1_ref.shape[0]
    O2 = w2_ref.shape[0]
    g = g_ref[0]
    nxy = nxy_ref[0]
    q = lax.dot_general(g.reshape(K * SCH, C), w1_ref[...],
                        (((1,), (1,)), ((), ())),
                        preferred_element_type=_F32)
    c = lax.dot_general(nxy, w1c_ref[...], (((1,), (1,)), ((), ())),
                        preferred_element_type=_F32)
    x1 = q.reshape(K, SCH, O) - c[None, :, :] + b1_ref[...][None]
    x1f = x1.reshape(K * SCH, O)
    h = jnp.maximum(x1f * s1_ref[...] + t1_ref[...], 0.0)
    x2 = lax.dot_general(h, w2_ref[...], (((1,), (1,)), ((), ())),
                         preferred_element_type=_F32) + b2_ref[...]
    sum_ref[0, 0] = jnp.sum(x2, axis=0, keepdims=True)
    sq_ref[0, 0] = jnp.sum(x2 * x2, axis=0, keepdims=True)
    x2r = x2.reshape(K, SCH, O2)
    mx_ref[0] = jnp.max(x2r, axis=0)
    mn_ref[0] = jnp.min(x2r, axis=0)


def _k6_call(G4, new_xy, W1, W1c, b1_2d, s1, t1, W2, b2_2d, SCH):
    B, K, NP, C = G4.shape
    O = W1.shape[0]
    O2 = W2.shape[0]
    NCH = NP // SCH
    return pl.pallas_call(
        _k6_body,
        grid=(B, NCH),
        in_specs=[
            pl.BlockSpec((1, K, SCH, C), lambda b, j: (b, 0, j, 0)),
            pl.BlockSpec((1, SCH, 2), lambda b, j: (b, j, 0)),
            pl.BlockSpec((O, C), lambda b, j: (0, 0)),
            pl.BlockSpec((O, 2), lambda b, j: (0, 0)),
            pl.BlockSpec((1, O), lambda b, j: (0, 0)),
            pl.BlockSpec((1, O), lambda b, j: (0, 0)),
            pl.BlockSpec((1, O), lambda b, j: (0, 0)),
            pl.BlockSpec((O2, O), lambda b, j: (0, 0)),
            pl.BlockSpec((1, O2), lambda b, j: (0, 0)),
        ],
        out_specs=[
            pl.BlockSpec((1, SCH, O2), lambda b, j: (b, j, 0)),
            pl.BlockSpec((1, SCH, O2), lambda b, j: (b, j, 0)),
            pl.BlockSpec((1, 1, 1, O2), lambda b, j: (b, j, 0, 0)),
            pl.BlockSpec((1, 1, 1, O2), lambda b, j: (b, j, 0, 0)),
        ],
        out_shape=[
            jax.ShapeDtypeStruct((B, NP, O2), _F32),
            jax.ShapeDtypeStruct((B, NP, O2), _F32),
            jax.ShapeDtypeStruct((B, NCH, 1, O2), _F32),
            jax.ShapeDtypeStruct((B, NCH, 1, O2), _F32),
        ],
    )(G4, new_xy, W1, W1c, b1_2d, s1, t1, W2, b2_2d)


def _k7_body(mx_ref, mn_ref, s2_ref, t2_ref, out_ref):
    mx = mx_ref[0]
    mn = mn_ref[0]
    a = s2_ref[...]
    t = t2_ref[...]
    choose = jnp.where(a >= 0.0, mx, mn)
    out_ref[0] = jnp.maximum(choose * a + t, 0.0)


def _k7_call(mx, mn, s2, t2):
    B, NP, O2 = mx.shape
    return pl.pallas_call(
        _k7_body,
        grid=(B,),
        in_specs=[
            pl.BlockSpec((1, NP, O2), lambda b: (b, 0, 0)),
            pl.BlockSpec((1, NP, O2), lambda b: (b, 0, 0)),
            pl.BlockSpec((1, O2), lambda b: (0, 0)),
            pl.BlockSpec((1, O2), lambda b: (0, 0)),
        ],
        out_specs=pl.BlockSpec((1, NP, O2), lambda b: (b, 0, 0)),
        out_shape=jax.ShapeDtypeStruct((B, NP, O2), _F32),
    )(mx, mn, s2, t2)


def kernel(xy, points, W1, b1, g1, be1, W2, b2, g2, be2):
    B, N, _ = xy.shape
    NP = N // 4
    K = 16
    EPS = 1e-5

    xx = xy[:, :, 0]
    yy = xy[:, :, 1]
    nx, ny = _fps_call(xx, yy, NP)
    new_xy = jnp.stack([nx, ny], axis=-1)

    idxT = _knn_call(xx[:, :, None], yy[:, :, None], nx[:, None, :],
                     ny[:, None, :], K)

    feats = jnp.concatenate([xy, points], axis=-1)
    C = feats.shape[-1]

    gidx = (idxT + (jnp.arange(B, dtype=_I32) * N)[:, None, None]).reshape(-1)
    G = _gather_rows(feats.reshape(B * N, C), gidx)
    G4 = G.reshape(B, K, NP, C)

    W1c = W1[:, :2]
    b1_2d = b1[None, :]
    SCH = 128
    M = jnp.float32(B * NP * K)

    sum1, sq1 = _k5_call(G4, new_xy, W1, W1c, b1_2d, SCH)
    mean1 = jnp.sum(sum1, axis=(0, 1)) / M
    var1 = jnp.maximum(jnp.sum(sq1, axis=(0, 1)) / M - mean1 * mean1, 0.0)
    s1 = g1[None, :] / jnp.sqrt(var1 + EPS)
    t1 = be1[None, :] - mean1 * s1

    mxp, mnp, sum2, sq2 = _k6_call(G4, new_xy, W1, W1c, b1_2d, s1, t1, W2,
                                   b2[None, :], SCH)
    mean2 = jnp.sum(sum2, axis=(0, 1)) / M
    var2 = jnp.maximum(jnp.sum(sq2, axis=(0, 1)) / M - mean2 * mean2, 0.0)
    s2 = g2[None, :] / jnp.sqrt(var2 + EPS)
    t2 = be2[None, :] - mean2 * s2

    out = _k7_call(mxp, mnp, s2, t2)
    return (new_xy, out)
